# baseline (speedup 1.0000x reference)
import functools

import jax
import jax.numpy as jnp
from jax import lax
from jax.experimental import pallas as pl
from jax.experimental.pallas import tpu as pltpu
from jax.experimental.pallas import tpu_sc as plsc

NC = 2
NS = 16
LN = 16


def _sc_aggregate(h, src, dst, w):
  N, H = h.shape
  P, E = src.shape
  NW = NC * NS
  CH = 128
  per_w_raw = -(-E // NW)
  NCH = -(-per_w_raw // CH)
  if NCH % 8:
    NCH += 8 - NCH % 8
  per_w = NCH * CH
  pad_total = NW * per_w - E

  def pad_edges(x, value):
    xp = jnp.pad(x, ((0, 0), (0, pad_total)), constant_values=value)
    return xp.reshape(P, NW, NCH, CH)

  src4 = pad_edges(src, 0)
  dst4 = pad_edges(dst, 0)
  w4 = pad_edges(w, 0.0)

  RA = (N // NS) // 8 * 8
  RB = N - RA * (NS - 1)
  assert RB % 8 == 0 and RB <= 2 * RA

  NB = 2
  GC = 40 if NCH % 40 == 0 else (20 if NCH % 20 == 0 else 8)
  assert NCH % GC == 0
  NG = NCH // GC
  mesh = plsc.VectorSubcoreMesh(core_axis_name="c", subcore_axis_name="s",
                                num_cores=NC, num_subcores=NS)

  def _scale_rows(rows_ref, w2_ref, ci):
    def body(g, carry):
      wv = w2_ref[ci, pl.ds(g * LN, LN)]
      for j in range(LN):
        wi = wv[j]
        row = g * LN + j
        for c in range(H // LN):
          sl = pl.ds(c * LN, LN)
          rows_ref[row, sl] = rows_ref[row, sl] * wi
      return carry
    lax.fori_loop(0, CH // LN, body, 0)

  @functools.partial(
      pl.kernel,
      out_type=jax.ShapeDtypeStruct((P, NC, N, H), jnp.float32),
      mesh=mesh,
      scratch_types=[
          pltpu.VMEM_SHARED((N, H), jnp.float32),
          pltpu.VMEM((GC, CH), jnp.int32),
          pltpu.VMEM((GC, CH), jnp.int32),
          pltpu.VMEM((GC, CH), jnp.float32),
          pltpu.VMEM((NB, CH, H), jnp.float32),
          pltpu.SemaphoreType.DMA((NB,)),
      ],
  )
  def body(h_hbm, src_hbm, dst_hbm, w_hbm, zero_hbm, parts_hbm,
           acc, src_b, dst_b, w_b, rows, gsem):
    cid = lax.axis_index("c")
    sid = lax.axis_index("s")
    wid = sid * NC + cid

    for p in range(P):
      @pl.when(sid < NS - 1)
      def _():
        pltpu.sync_copy(zero_hbm.at[pl.ds(0, RA)],
                        acc.at[pl.ds(sid * RA, RA)])

      @pl.when(sid == NS - 1)
      def _():
        pltpu.sync_copy(zero_hbm, acc.at[pl.ds((NS - 1) * RA, RB)])

      plsc.subcore_barrier()

      def gather(j, b):
        pltpu.async_copy(h_hbm.at[src_b.at[j]], rows.at[b], gsem.at[b])

      def group(g, carry):
        sl = pl.ds(g * GC, GC)
        pltpu.sync_copy(src_hbm.at[p, wid, sl], src_b)
        pltpu.sync_copy(dst_hbm.at[p, wid, sl], dst_b)
        pltpu.sync_copy(w_hbm.at[p, wid, sl], w_b)
        gather(0, 0)

        def pair(k, carry2):
          for jj in range(NB):
            j = k * NB + jj
            pltpu.make_async_copy(h_hbm.at[src_b.at[j]], rows.at[jj],
                                  gsem.at[jj]).wait()

            @pl.when(j + 1 < GC)
            def _():
              gather(j + 1, (jj + 1) % NB)
            _scale_rows(rows.at[jj], w_b, j)
            pltpu.sync_copy(rows.at[jj], acc.at[dst_b.at[j]], add=True)
          return carry2
        lax.fori_loop(0, GC // NB, pair, 0)
        return carry
      lax.fori_loop(0, NG, group, 0)

      plsc.subcore_barrier()

      @pl.when(sid < NS - 1)
      def _():
        pltpu.sync_copy(
            acc.at[pl.ds(sid * RA, RA)],
            parts_hbm.at[p, cid, pl.ds(sid * RA, RA)])

      @pl.when(sid == NS - 1)
      def _():
        pltpu.sync_copy(
            acc.at[pl.ds((NS - 1) * RA, RB)],
            parts_hbm.at[p, cid, pl.ds((NS - 1) * RA, RB)])

      plsc.subcore_barrier()

  zeros_slab = jnp.zeros((RB, H), dtype=jnp.float32)
  return body(h, src4, dst4, w4, zeros_slab)


def _tc_transform(parts, W, b, prelu_a, fc_W, fc_b):
  P, _, N, H = parts.shape
  BN = 1000 if N % 1000 == 0 else N
  nb = N // BN

  def body(parts_ref, W_ref, b_ref, a_ref, fcW_ref, fcb_ref,
           emb_ref, sacc_ref):
    i = pl.program_id(0)

    @pl.when(i == 0)
    def _():
      sacc_ref[...] = jnp.zeros_like(sacc_ref)

    for p in range(P):
      agg = parts_ref[p, 0] + parts_ref[p, 1]
      fts = lax.dot_general(agg, W_ref[p], (((1,), (1,)), ((), ())),
                            preferred_element_type=jnp.float32)
      x = fts + b_ref[p:p + 1, :]
      a = a_ref[0, p]
      e = jnp.where(x > 0, x, a * x)
      emb_ref[p] = e
      t = jnp.tanh(
          lax.dot_general(e, fcW_ref[...], (((1,), (1,)), ((), ())),
                          preferred_element_type=jnp.float32)
          + fcb_ref[...])
      sacc_ref[p:p + 1, :] += jnp.sum(t, axis=0, keepdims=True)

  emb, sacc = pl.pallas_call(
      body,
      grid=(nb,),
      in_specs=[
          pl.BlockSpec((P, 2, BN, H), lambda i: (0, 0, i, 0)),
          pl.BlockSpec((P, H, H), lambda i: (0, 0, 0)),
          pl.BlockSpec((P, H), lambda i: (0, 0)),
          pl.BlockSpec(memory_space=pltpu.SMEM),
          pl.BlockSpec((H, H), lambda i: (0, 0)),
          pl.BlockSpec((1, H), lambda i: (0, 0)),
      ],
      out_specs=[
          pl.BlockSpec((P, BN, H), lambda i: (0, i, 0)),
          pl.BlockSpec((P, H), lambda i: (0, 0)),
      ],
      out_shape=[
          jax.ShapeDtypeStruct((P, N, H), jnp.float32),
          jax.ShapeDtypeStruct((P, H), jnp.float32),
      ],
  )(parts, W, b, prelu_a.reshape(1, P), fc_W, fc_b.reshape(1, H))
  return emb, sacc


def _tc_combine(emb, sacc, att, n_nodes):
  P, N, H = emb.shape
  BN = 1000 if N % 1000 == 0 else N
  nb = N // BN

  def body(emb_ref, sacc_ref, att_ref, z_ref):
    logits = [
        jnp.sum(att_ref[...] * sacc_ref[p:p + 1, :], axis=1, keepdims=True)
        / n_nodes
        for p in range(P)
    ]
    m = logits[0]
    for p in range(1, P):
      m = jnp.maximum(m, logits[p])
    exps = [jnp.exp(l - m) for l in logits]
    se = exps[0]
    for p in range(1, P):
      se = se + exps[p]
    acc = (exps[0] / se) * emb_ref[0]
    for p in range(1, P):
      acc = acc + (exps[p] / se) * emb_ref[p]
    z_ref[...] = acc

  return pl.pallas_call(
      body,
      grid=(nb,),
      in_specs=[
          pl.BlockSpec((P, BN, H), lambda i: (0, i, 0)),
          pl.BlockSpec((P, H), lambda i: (0, 0)),
          pl.BlockSpec((1, H), lambda i: (0, 0)),
      ],
      out_specs=pl.BlockSpec((BN, H), lambda i: (i, 0)),
      out_shape=jax.ShapeDtypeStruct((N, H), jnp.float32),
  )(emb, sacc, att.reshape(1, H))


def kernel(h, edge_index, edge_weight, W, b, prelu_a, fc_W, fc_b, att):
  N, H = h.shape
  P = edge_index.shape[0]
  dst = edge_index[:, 0, :]
  src = edge_index[:, 1, :]
  parts = _sc_aggregate(h, src, dst, edge_weight)
  emb, sacc = _tc_transform(parts, W, b, prelu_a, fc_W, fc_b)
  return _tc_combine(emb, sacc, att, float(N))
